# Initial kernel scaffold; baseline (speedup 1.0000x reference)
import jax
import jax.numpy as jnp
from jax.experimental import pallas as pl


def kernel(input_ids, table):
    raise NotImplementedError("write your pallas kernel here")


# speedup vs baseline: 4.0804x; 4.0804x over previous
import functools

import jax
import jax.numpy as jnp
from jax import lax
from jax.experimental import pallas as pl
from jax.experimental.pallas import tpu as pltpu
from jax.experimental.pallas import tpu_sc as plsc

VOCAB = 100000
EMB_DIM = 64
BATCH = 4096
SEQ = 50

_INFO = plsc.get_sparse_core_info()
NC = _INFO.num_cores
NS = _INFO.num_subcores
NW = NC * NS
TOTAL = BATCH * SEQ
B_PER_W = TOTAL // NW
CHUNK = 128
N_CHUNKS = B_PER_W // CHUNK

_MESH = plsc.VectorSubcoreMesh(core_axis_name="c", subcore_axis_name="s")


@functools.partial(
    pl.kernel,
    out_type=jax.ShapeDtypeStruct((TOTAL, EMB_DIM), jnp.float32),
    mesh=_MESH,
    scratch_types=[
        pltpu.VMEM((N_CHUNKS, CHUNK), jnp.int32),
        pltpu.VMEM((CHUNK, EMB_DIM), jnp.float32),
        pltpu.SemaphoreType.DMA,
    ],
    compiler_params=pltpu.CompilerParams(use_tc_tiling_on_sc=False),
)
def _gather_kernel(idx_hbm, table_hbm, out_hbm, idx_v, rows_v, sem):
    wid = lax.axis_index("s") * NC + lax.axis_index("c")
    base = wid * B_PER_W
    pltpu.sync_copy(idx_hbm.at[wid], idx_v)

    def body(j, _):
        pltpu.async_copy(table_hbm.at[idx_v.at[j]], rows_v, sem).wait()
        pltpu.sync_copy(rows_v, out_hbm.at[pl.ds(base + j * CHUNK, CHUNK)])
        return 0

    lax.fori_loop(0, N_CHUNKS, body, 0)


def kernel(input_ids, table):
    idx = input_ids.reshape(NW, N_CHUNKS, CHUNK).astype(jnp.int32)
    out = _gather_kernel(idx, table)
    return out.reshape(BATCH, SEQ, EMB_DIM)


# speedup vs baseline: 4.6492x; 1.1394x over previous
import functools

import jax
import jax.numpy as jnp
from jax import lax
from jax.experimental import pallas as pl
from jax.experimental.pallas import tpu as pltpu
from jax.experimental.pallas import tpu_sc as plsc

VOCAB = 100000
EMB_DIM = 64
BATCH = 4096
SEQ = 50

_INFO = plsc.get_sparse_core_info()
NC = _INFO.num_cores
NS = _INFO.num_subcores
NW = NC * NS
TOTAL = BATCH * SEQ
B_PER_W = TOTAL // NW
CHUNK = 128
N_CHUNKS = B_PER_W // CHUNK

_MESH = plsc.VectorSubcoreMesh(core_axis_name="c", subcore_axis_name="s")

G = 5
N_GROUPS = N_CHUNKS // G
G_ROWS = G * CHUNK


@functools.partial(
    pl.kernel,
    out_type=jax.ShapeDtypeStruct((TOTAL, EMB_DIM), jnp.float32),
    mesh=_MESH,
    scratch_types=[
        pltpu.VMEM((N_CHUNKS, CHUNK), jnp.int32),
        pltpu.VMEM((G_ROWS, EMB_DIM), jnp.float32),
        pltpu.VMEM((G_ROWS, EMB_DIM), jnp.float32),
        pltpu.SemaphoreType.DMA,
        pltpu.SemaphoreType.DMA,
        pltpu.SemaphoreType.DMA,
    ],
    compiler_params=pltpu.CompilerParams(use_tc_tiling_on_sc=False),
)
def _gather_kernel(idx_hbm, table_hbm, out_hbm, idx_v, buf0, buf1, sem_g,
                   sem_w0, sem_w1):
    wid = lax.axis_index("s") * NC + lax.axis_index("c")
    base = wid * B_PER_W
    pltpu.sync_copy(idx_hbm.at[wid], idx_v)

    bufs = (buf0, buf1)
    sems_w = (sem_w0, sem_w1)

    def fire_gathers(t, buf):
        return [
            pltpu.async_copy(
                table_hbm.at[idx_v.at[t * G + g]],
                buf.at[pl.ds(g * CHUNK, CHUNK)],
                sem_g,
            )
            for g in range(G)
        ]

    pending_g = fire_gathers(0, bufs[0])
    pending_w = [None, None]
    for t in range(N_GROUPS):
        p = t % 2
        q = 1 - p
        if t + 1 < N_GROUPS:
            if pending_w[q] is not None:
                pending_w[q].wait()
                pending_w[q] = None
            next_g = fire_gathers(t + 1, bufs[q])
        for d in pending_g:
            d.wait()
        pending_w[p] = pltpu.async_copy(
            bufs[p], out_hbm.at[pl.ds(base + t * G_ROWS, G_ROWS)], sems_w[p]
        )
        if t + 1 < N_GROUPS:
            pending_g = next_g
    for d in pending_w:
        if d is not None:
            d.wait()


def kernel(input_ids, table):
    idx = input_ids.reshape(NW, N_CHUNKS, CHUNK).astype(jnp.int32)
    out = _gather_kernel(idx, table)
    return out.reshape(BATCH, SEQ, EMB_DIM)
